# baseline (speedup 1.0000x reference)
import jax
import jax.numpy as jnp
from jax import lax
from jax.experimental import pallas as pl
from jax.experimental.pallas import tpu as pltpu
from jax.experimental.pallas import tpu_sc as plsc

VOCAB = 1000000
EMB = 64
B = 4096
L = 200

NC = 2
NS = 16
NW = NC * NS


NBLK = 246
LASTC0 = 999808
TAIL = VOCAB - 7812 * 128


def _iota16(mult, base):
    return lax.iota(jnp.int32, 16) * mult + base


def _a_transpose(blk, t, nj=128):
    def body(jq, c):
        for dj in range(4):
            j = jq * 4 + dj
            cols = jnp.full((16,), j, dtype=jnp.int32)
            for e0 in range(4):
                v = plsc.load_gather(blk, [_iota16(1, e0 * 16), cols])
                t[j, pl.ds(e0 * 16, 16)] = v
        return c

    lax.fori_loop(0, nj // 4, body, 0)


def _a_body(tT, tail, P, blk0, blk1, t0, t1, tl_v, rs0, rs1, ws0, ws1):
    w = lax.axis_index("s") * NC + lax.axis_index("c")
    blks = (blk0, blk1)
    ts = (t0, t1)
    rs = (rs0, rs1)
    ws = (ws0, ws1)

    def c0_of(k):
        return pl.multiple_of(jnp.minimum((k * 32 + w) * 128, LASTC0), 128)

    def fire_read(k, b):
        pltpu.async_copy(tT.at[:, pl.ds(c0_of(k), 128)], blks[b], rs[b])

    def wait_read(k, b):
        pltpu.make_async_copy(
            tT.at[:, pl.ds(c0_of(k), 128)], blks[b], rs[b]
        ).wait()

    def fire_write(k, b):
        pltpu.async_copy(ts[b], P.at[pl.ds(c0_of(k), 128)], ws[b])

    def wait_write(k, b):
        pltpu.make_async_copy(ts[b], P.at[pl.ds(c0_of(k), 128)], ws[b]).wait()

    fire_read(0, 0)
    fire_read(1, 1)
    wait_read(0, 0)
    _a_transpose(blk0, t0)
    fire_write(0, 0)
    fire_read(2, 0)
    wait_read(1, 1)
    _a_transpose(blk1, t1)
    fire_write(1, 1)
    fire_read(3, 1)

    def outer(o, carry):
        for b in range(2):
            k = 2 * o + b
            wait_read(k, b)
            wait_write(k - 2, b)
            _a_transpose(blks[b], ts[b])
            fire_write(k, b)
            fire_read(k + 2, b)
        return carry

    lax.fori_loop(1, NBLK // 2 - 1, outer, 0)

    for b in range(2):
        k = NBLK - 2 + b
        wait_read(k, b)
        wait_write(k - 2, b)
        _a_transpose(blks[b], ts[b])
        fire_write(k, b)
    wait_write(NBLK - 2, 0)
    wait_write(NBLK - 1, 1)

    @pl.when(w == 0)
    def _tail():
        pltpu.sync_copy(tail, tl_v)
        _a_transpose(tl_v, t0, nj=TAIL)
        pltpu.sync_copy(t0.at[pl.ds(0, TAIL)], P.at[pl.ds(7812 * 128, TAIL)])


PER_W = L


ROWS16 = None


def _b_transpose(g, t):
    def body(eq, c):
        for de in range(4):
            e = eq * 4 + de
            e8 = lax.shift_right_logical(e, 3)
            e0 = lax.bitwise_and(e, 7)
            cols = jnp.full((16,), e, dtype=jnp.int32)
            for j in range(8):
                v = plsc.load_gather(g, [_iota16(1, j * 16), cols])
                t[e8, e0, pl.ds(j * 16, 16)] = v
        return c

    lax.fori_loop(0, 16, body, 0)


def _b_body(idx2, P2, out, slab, c0, c1, g0, g1, t0, t1, gs0, gs1, ss0, ss1):
    w = lax.axis_index("s") * NC + lax.axis_index("c")
    cs = (c0, c1)
    gs = (g0, g1)
    ts = (t0, t1)
    gsem = (gs0, gs1)
    ssem = (ss0, ss1)

    pltpu.sync_copy(idx2.at[pl.ds(w * PER_W, PER_W)], slab)
    bases = [_iota16(L, j8 * 16 * L) for j8 in range(8)]

    def assemble(l, b):
        for j8 in range(8):
            pos = bases[j8] + l
            v = plsc.load_gather(
                slab, [lax.shift_right_logical(pos, 7), lax.bitwise_and(pos, 127)]
            )
            cs[b][pl.ds(j8 * 16, 16)] = v

    def fire_gather(b):
        pltpu.async_copy(P2.at[cs[b]], gs[b], gsem[b])

    def wait_gather(b):
        pltpu.make_async_copy(P2.at[cs[b]], gs[b], gsem[b]).wait()

    def fire_stores(l, b):
        for e8 in range(8):
            pltpu.async_copy(ts[b].at[e8], out.at[l, e8, w], ssem[b])

    def wait_stores(l, b):
        for e8 in range(8):
            pltpu.make_async_copy(ts[b].at[e8], out.at[l, e8, w], ssem[b]).wait()

    def step(l, b, first):
        if l + 1 < L:
            assemble(l + 1, 1 - b)
            fire_gather(1 - b)
        wait_gather(b)
        if not first:
            wait_stores(l - 2, b)
        _b_transpose(gs[b], ts[b])
        fire_stores(l, b)

    assemble(0, 0)
    fire_gather(0)
    step(0, 0, True)
    step(1, 1, True)

    def outer(o, carry):
        for b in range(2):
            l = 2 * o + b
            assemble(l + 1, 1 - b)
            fire_gather(1 - b)
            wait_gather(b)
            wait_stores(l - 2, b)
            _b_transpose(gs[b], ts[b])
            fire_stores(l, b)
        return carry

    lax.fori_loop(1, L // 2 - 1, outer, 0)

    step(L - 2, 0, False)
    step(L - 1, 1, False)
    wait_stores(L - 2, 0)
    wait_stores(L - 1, 1)


@jax.jit
def kernel(idx, table):
    mesh = plsc.VectorSubcoreMesh(
        core_axis_name="c", subcore_axis_name="s", num_cores=NC, num_subcores=NS
    )
    P = pl.kernel(
        _a_body,
        out_type=jax.ShapeDtypeStruct((VOCAB, 128), jnp.float32),
        mesh=mesh,
        scratch_types=[
            pltpu.VMEM((64, 128), jnp.float32),
            pltpu.VMEM((64, 128), jnp.float32),
            pltpu.VMEM((128, 128), jnp.float32),
            pltpu.VMEM((128, 128), jnp.float32),
            pltpu.VMEM((64, TAIL), jnp.float32),
            pltpu.SemaphoreType.DMA,
            pltpu.SemaphoreType.DMA,
            pltpu.SemaphoreType.DMA,
            pltpu.SemaphoreType.DMA,
        ],
        compiler_params=pltpu.CompilerParams(
            use_tc_tiling_on_sc=True, needs_layout_passes=False
        ),
    )(table.T, table[7812 * 128 :].T)

    P2 = P.reshape(2 * VOCAB, EMB)
    idx2 = idx.reshape(B * L // 128, 128).astype(jnp.int32) * 2
    outL = pl.kernel(
        _b_body,
        out_type=jax.ShapeDtypeStruct((L, 8, 32, 8, 128), jnp.float32),
        mesh=mesh,
        scratch_types=[
            pltpu.VMEM((PER_W, 128), jnp.int32),
            pltpu.VMEM((128,), jnp.int32),
            pltpu.VMEM((128,), jnp.int32),
            pltpu.VMEM((128, 64), jnp.float32),
            pltpu.VMEM((128, 64), jnp.float32),
            pltpu.VMEM((8, 8, 128), jnp.float32),
            pltpu.VMEM((8, 8, 128), jnp.float32),
            pltpu.SemaphoreType.DMA,
            pltpu.SemaphoreType.DMA,
            pltpu.SemaphoreType.DMA,
            pltpu.SemaphoreType.DMA,
        ],
        compiler_params=pltpu.CompilerParams(
            use_tc_tiling_on_sc=False, needs_layout_passes=False
        ),
    )(idx2, P2)

    return outL.transpose(2, 4, 0, 1, 3).reshape(B, L, EMB)


# speedup vs baseline: 1.8254x; 1.8254x over previous
import jax
import jax.numpy as jnp
from jax import lax
from jax.experimental import pallas as pl
from jax.experimental.pallas import tpu as pltpu
from jax.experimental.pallas import tpu_sc as plsc

VOCAB = 1000000
EMB = 64
B = 4096
L = 200

NC = 2
NS = 16
NW = NC * NS


NBLK = 246
LASTC0 = 999808
TAIL = VOCAB - 7812 * 128


def _iota16(mult, base):
    return lax.iota(jnp.int32, 16) * mult + base


def _a_transpose(blk, t, nj=128):
    @plsc.parallel_loop(0, nj, unroll=8)
    def _(j):
        cols = jnp.full((16,), j, dtype=jnp.int32)
        for e0 in range(4):
            v = plsc.load_gather(blk, [_iota16(1, e0 * 16), cols])
            t[j, pl.ds(e0 * 16, 16)] = v


def _a_body(tT, tail, P, blk0, blk1, t0, t1, tl_v, rs0, rs1, ws0, ws1):
    w = lax.axis_index("s") * NC + lax.axis_index("c")
    blks = (blk0, blk1)
    ts = (t0, t1)
    rs = (rs0, rs1)
    ws = (ws0, ws1)

    def c0_of(k):
        return pl.multiple_of(jnp.minimum((k * 32 + w) * 128, LASTC0), 128)

    def fire_read(k, b):
        pltpu.async_copy(tT.at[:, pl.ds(c0_of(k), 128)], blks[b], rs[b])

    def wait_read(k, b):
        pltpu.make_async_copy(
            tT.at[:, pl.ds(c0_of(k), 128)], blks[b], rs[b]
        ).wait()

    def fire_write(k, b):
        pltpu.async_copy(ts[b], P.at[pl.ds(c0_of(k), 128)], ws[b])

    def wait_write(k, b):
        pltpu.make_async_copy(ts[b], P.at[pl.ds(c0_of(k), 128)], ws[b]).wait()

    fire_read(0, 0)
    fire_read(1, 1)
    wait_read(0, 0)
    _a_transpose(blk0, t0)
    fire_write(0, 0)
    fire_read(2, 0)
    wait_read(1, 1)
    _a_transpose(blk1, t1)
    fire_write(1, 1)
    fire_read(3, 1)

    def outer(o, carry):
        for b in range(2):
            k = 2 * o + b
            wait_read(k, b)
            wait_write(k - 2, b)
            _a_transpose(blks[b], ts[b])
            fire_write(k, b)
            fire_read(k + 2, b)
        return carry

    lax.fori_loop(1, NBLK // 2 - 1, outer, 0)

    for b in range(2):
        k = NBLK - 2 + b
        wait_read(k, b)
        wait_write(k - 2, b)
        _a_transpose(blks[b], ts[b])
        fire_write(k, b)
    wait_write(NBLK - 2, 0)
    wait_write(NBLK - 1, 1)

    @pl.when(w == 0)
    def _tail():
        pltpu.sync_copy(tail, tl_v)
        _a_transpose(tl_v, t0, nj=TAIL)
        pltpu.sync_copy(t0.at[pl.ds(0, TAIL)], P.at[pl.ds(7812 * 128, TAIL)])


PER_W = L


ROWS16 = None


def _b_transpose(g, t):
    @plsc.parallel_loop(0, 64, unroll=8)
    def _(e):
        e8 = lax.shift_right_logical(e, 3)
        e0 = lax.bitwise_and(e, 7)
        cols = jnp.full((16,), e, dtype=jnp.int32)
        for j in range(8):
            v = plsc.load_gather(g, [_iota16(1, j * 16), cols])
            t[e8, e0, pl.ds(j * 16, 16)] = v


def _b_body(idx2, P2, out, slab, c0, c1, g0, g1, t0, t1, gs0, gs1, ss0, ss1):
    w = lax.axis_index("s") * NC + lax.axis_index("c")
    cs = (c0, c1)
    gs = (g0, g1)
    ts = (t0, t1)
    gsem = (gs0, gs1)
    ssem = (ss0, ss1)

    pltpu.sync_copy(idx2.at[pl.ds(w * PER_W, PER_W)], slab)
    bases = [_iota16(L, j8 * 16 * L) for j8 in range(8)]

    def assemble(l, b):
        for j8 in range(8):
            pos = bases[j8] + l
            v = plsc.load_gather(
                slab, [lax.shift_right_logical(pos, 7), lax.bitwise_and(pos, 127)]
            )
            cs[b][pl.ds(j8 * 16, 16)] = v

    def fire_gather(b):
        pltpu.async_copy(P2.at[cs[b]], gs[b], gsem[b])

    def wait_gather(b):
        pltpu.make_async_copy(P2.at[cs[b]], gs[b], gsem[b]).wait()

    def fire_stores(l, b):
        for e8 in range(8):
            pltpu.async_copy(ts[b].at[e8], out.at[l, e8, w], ssem[b])

    def wait_stores(l, b):
        for e8 in range(8):
            pltpu.make_async_copy(ts[b].at[e8], out.at[l, e8, w], ssem[b]).wait()

    def step(l, b, first):
        if l + 1 < L:
            assemble(l + 1, 1 - b)
            fire_gather(1 - b)
        wait_gather(b)
        if not first:
            wait_stores(l - 2, b)
        _b_transpose(gs[b], ts[b])
        fire_stores(l, b)

    assemble(0, 0)
    fire_gather(0)
    step(0, 0, True)
    step(1, 1, True)

    def outer(o, carry):
        for b in range(2):
            l = 2 * o + b
            assemble(l + 1, 1 - b)
            fire_gather(1 - b)
            wait_gather(b)
            wait_stores(l - 2, b)
            _b_transpose(gs[b], ts[b])
            fire_stores(l, b)
        return carry

    lax.fori_loop(1, L // 2 - 1, outer, 0)

    step(L - 2, 0, False)
    step(L - 1, 1, False)
    wait_stores(L - 2, 0)
    wait_stores(L - 1, 1)


@jax.jit
def kernel(idx, table):
    mesh = plsc.VectorSubcoreMesh(
        core_axis_name="c", subcore_axis_name="s", num_cores=NC, num_subcores=NS
    )
    P = pl.kernel(
        _a_body,
        out_type=jax.ShapeDtypeStruct((VOCAB, 128), jnp.float32),
        mesh=mesh,
        scratch_types=[
            pltpu.VMEM((64, 128), jnp.float32),
            pltpu.VMEM((64, 128), jnp.float32),
            pltpu.VMEM((128, 128), jnp.float32),
            pltpu.VMEM((128, 128), jnp.float32),
            pltpu.VMEM((64, TAIL), jnp.float32),
            pltpu.SemaphoreType.DMA,
            pltpu.SemaphoreType.DMA,
            pltpu.SemaphoreType.DMA,
            pltpu.SemaphoreType.DMA,
        ],
        compiler_params=pltpu.CompilerParams(
            use_tc_tiling_on_sc=True, needs_layout_passes=False
        ),
    )(table.T, table[7812 * 128 :].T)

    P2 = P.reshape(2 * VOCAB, EMB)
    idx2 = idx.reshape(B * L // 128, 128).astype(jnp.int32) * 2
    outL = pl.kernel(
        _b_body,
        out_type=jax.ShapeDtypeStruct((L, 8, 32, 8, 128), jnp.float32),
        mesh=mesh,
        scratch_types=[
            pltpu.VMEM((PER_W, 128), jnp.int32),
            pltpu.VMEM((128,), jnp.int32),
            pltpu.VMEM((128,), jnp.int32),
            pltpu.VMEM((128, 64), jnp.float32),
            pltpu.VMEM((128, 64), jnp.float32),
            pltpu.VMEM((8, 8, 128), jnp.float32),
            pltpu.VMEM((8, 8, 128), jnp.float32),
            pltpu.SemaphoreType.DMA,
            pltpu.SemaphoreType.DMA,
            pltpu.SemaphoreType.DMA,
            pltpu.SemaphoreType.DMA,
        ],
        compiler_params=pltpu.CompilerParams(
            use_tc_tiling_on_sc=False, needs_layout_passes=False
        ),
    )(idx2, P2)

    return outL.transpose(2, 4, 0, 1, 3).reshape(B, L, EMB)


# speedup vs baseline: 2.6524x; 1.4531x over previous
import jax
import jax.numpy as jnp
from jax import lax
from jax.experimental import pallas as pl
from jax.experimental.pallas import tpu as pltpu
from jax.experimental.pallas import tpu_sc as plsc

VOCAB = 1000000
EMB = 64
B = 4096
L = 200

NC = 2
NS = 16
NW = NC * NS


NBLK = 246
LASTC0 = 999808
TAIL = VOCAB - 7812 * 128


def _iota16(mult, base):
    return lax.iota(jnp.int32, 16) * mult + base


def _perms():
    iota = lax.iota(jnp.int32, 16)
    return iota, [lax.bitwise_and(iota + k, 15) for k in range(16)]


def _a_transpose(blk, t, nj=128):
    iota, perm = _perms()
    @plsc.parallel_loop(0, nj // 16, unroll=1)
    def _(jq):
        j_vec = iota + jq * 16
        for eq in range(4):
            for k in range(16):
                e_vec = perm[k] + eq * 16
                v = plsc.load_gather(blk, [e_vec, j_vec])
                plsc.store_scatter(t, [j_vec, e_vec], v)


def _a_body(tT, tail, P, blk0, blk1, t0, t1, tl_v, rs0, rs1, ws0, ws1):
    w = lax.axis_index("s") * NC + lax.axis_index("c")
    blks = (blk0, blk1)
    ts = (t0, t1)
    rs = (rs0, rs1)
    ws = (ws0, ws1)

    def c0_of(k):
        return pl.multiple_of(jnp.minimum((k * 32 + w) * 128, LASTC0), 128)

    def fire_read(k, b):
        pltpu.async_copy(tT.at[:, pl.ds(c0_of(k), 128)], blks[b], rs[b])

    def wait_read(k, b):
        pltpu.make_async_copy(
            tT.at[:, pl.ds(c0_of(k), 128)], blks[b], rs[b]
        ).wait()

    def fire_write(k, b):
        pltpu.async_copy(ts[b], P.at[pl.ds(c0_of(k), 128)], ws[b])

    def wait_write(k, b):
        pltpu.make_async_copy(ts[b], P.at[pl.ds(c0_of(k), 128)], ws[b]).wait()

    def outer(o, carry):
        for b in range(2):
            k = 2 * o + b

            @pl.when(k < NBLK)
            def _fire():
                fire_read(k, b)

            kp = k - 1
            bp = 1 - b

            @pl.when(jnp.logical_and(kp >= 0, kp < NBLK))
            def _retire():
                wait_read(kp, bp)

                @pl.when(kp >= 2)
                def _():
                    wait_write(kp - 2, bp)

                _a_transpose(blks[bp], ts[bp])
                fire_write(kp, bp)

        return carry

    lax.fori_loop(0, NBLK // 2 + 1, outer, 0)
    wait_write(NBLK - 2, 0)
    wait_write(NBLK - 1, 1)

    @pl.when(w == 0)
    def _tail():
        pltpu.sync_copy(tail, tl_v)
        _a_transpose(tl_v, t0, nj=TAIL)
        pltpu.sync_copy(t0.at[pl.ds(0, TAIL)], P.at[pl.ds(7812 * 128, TAIL)])


PER_W = L


ROWS16 = None


def _b_transpose(g, t):
    iota, perm = _perms()

    @plsc.parallel_loop(0, 8, unroll=1)
    def _(bq):
        for eq in range(4):
            e_vec = iota + eq * 16
            for k in range(16):
                b_vec = perm[k] + bq * 16
                v = plsc.load_gather(g, [b_vec, e_vec])
                plsc.store_scatter(t, [(e_vec * 128) + b_vec], v)


def _b_body(idx2, P2, out, slab, c0, c1, g0, g1, t0, t1, gs0, gs1, ss0, ss1):
    w = lax.axis_index("s") * NC + lax.axis_index("c")
    cs = (c0, c1)
    gs = (g0, g1)
    ts = (t0, t1)
    gsem = (gs0, gs1)
    ssem = (ss0, ss1)

    pltpu.sync_copy(idx2.at[pl.ds(w * PER_W, PER_W)], slab)
    bases = [_iota16(L, j8 * 16 * L) for j8 in range(8)]

    def assemble(l, b):
        for j8 in range(8):
            pos = bases[j8] + l
            v = plsc.load_gather(
                slab, [lax.shift_right_logical(pos, 7), lax.bitwise_and(pos, 127)]
            )
            cs[b][pl.ds(j8 * 16, 16)] = v

    def fire_gather(b):
        pltpu.async_copy(P2.at[cs[b]], gs[b], gsem[b])

    def wait_gather(b):
        pltpu.make_async_copy(P2.at[cs[b]], gs[b], gsem[b]).wait()

    def fire_stores(l, b):
        for e8 in range(8):
            pltpu.async_copy(
                ts[b].at[pl.ds(e8 * 1024, 1024)], out.at[l, e8, w], ssem[b]
            )

    def wait_stores(l, b):
        for e8 in range(8):
            pltpu.make_async_copy(
                ts[b].at[pl.ds(e8 * 1024, 1024)], out.at[l, e8, w], ssem[b]
            ).wait()

    def outer(o, carry):
        for b in range(2):
            l = 2 * o + b

            @pl.when(l < L)
            def _fire():
                assemble(l, b)
                fire_gather(b)

            lp = l - 1
            bp = 1 - b

            @pl.when(jnp.logical_and(lp >= 0, lp < L))
            def _retire():
                wait_gather(bp)

                @pl.when(lp >= 2)
                def _():
                    wait_stores(lp - 2, bp)

                _b_transpose(gs[bp], ts[bp])
                fire_stores(lp, bp)

        return carry

    lax.fori_loop(0, L // 2 + 1, outer, 0)

    wait_stores(L - 2, 0)
    wait_stores(L - 1, 1)


@jax.jit
def kernel(idx, table):
    mesh = plsc.VectorSubcoreMesh(
        core_axis_name="c", subcore_axis_name="s", num_cores=NC, num_subcores=NS
    )
    P = pl.kernel(
        _a_body,
        out_type=jax.ShapeDtypeStruct((VOCAB, 128), jnp.float32),
        mesh=mesh,
        scratch_types=[
            pltpu.VMEM((64, 128), jnp.float32),
            pltpu.VMEM((64, 128), jnp.float32),
            pltpu.VMEM((128, 128), jnp.float32),
            pltpu.VMEM((128, 128), jnp.float32),
            pltpu.VMEM((64, TAIL), jnp.float32),
            pltpu.SemaphoreType.DMA,
            pltpu.SemaphoreType.DMA,
            pltpu.SemaphoreType.DMA,
            pltpu.SemaphoreType.DMA,
        ],
        compiler_params=pltpu.CompilerParams(
            use_tc_tiling_on_sc=True, needs_layout_passes=False
        ),
    )(table.T, table[7812 * 128 :].T)

    P2 = P.reshape(2 * VOCAB, EMB)
    idx2 = idx.reshape(B * L // 128, 128).astype(jnp.int32) * 2
    outL = pl.kernel(
        _b_body,
        out_type=jax.ShapeDtypeStruct((L, 8, 32, 1024), jnp.float32),
        mesh=mesh,
        scratch_types=[
            pltpu.VMEM((PER_W, 128), jnp.int32),
            pltpu.VMEM((128,), jnp.int32),
            pltpu.VMEM((128,), jnp.int32),
            pltpu.VMEM((128, 64), jnp.float32),
            pltpu.VMEM((128, 64), jnp.float32),
            pltpu.VMEM((8192,), jnp.float32),
            pltpu.VMEM((8192,), jnp.float32),
            pltpu.SemaphoreType.DMA,
            pltpu.SemaphoreType.DMA,
            pltpu.SemaphoreType.DMA,
            pltpu.SemaphoreType.DMA,
        ],
        compiler_params=pltpu.CompilerParams(
            use_tc_tiling_on_sc=False, needs_layout_passes=False
        ),
    )(idx2, P2)

    outL5 = outL.reshape(L, 8, 32, 8, 128)
    return outL5.transpose(2, 4, 0, 1, 3).reshape(B, L, EMB)


# speedup vs baseline: 6.0062x; 2.2645x over previous
import jax
import jax.numpy as jnp
from jax import lax
from jax.experimental import pallas as pl
from jax.experimental.pallas import tpu as pltpu
from jax.experimental.pallas import tpu_sc as plsc

VOCAB = 1000000
EMB = 64
B = 4096
L = 200

NC = 2
NS = 16
NW = NC * NS


NBLK = 248
LASTC0 = 999808
TAIL = VOCAB - 7812 * 128


def _iota16(mult, base):
    return lax.iota(jnp.int32, 16) * mult + base


def _perms():
    iota = lax.iota(jnp.int32, 16)
    return iota, [lax.bitwise_and(iota + k, 15) for k in range(16)]


def _a_transpose(blk, t, nj=128):
    iota, _ = _perms()

    @plsc.parallel_loop(0, (nj // 16) * 16, unroll=2)
    def _(i):
        jq = lax.shift_right_logical(i, 4)
        k = lax.bitwise_and(i, 15)
        j_vec = iota + jq * 16
        pv = lax.bitwise_and(iota + k, 15)
        for eq in range(4):
            e_vec = pv + eq * 16
            v = plsc.load_gather(blk, [e_vec, j_vec])
            plsc.store_scatter(t, [j_vec, e_vec], v)


def _a_body(tT, tail, P, blk0, blk1, blk2, blk3, t0, t1, t2, t3, tl_v,
            rs0, rs1, rs2, rs3, ws0, ws1, ws2, ws3):
    w = lax.axis_index("s") * NC + lax.axis_index("c")
    blks = (blk0, blk1, blk2, blk3)
    ts = (t0, t1, t2, t3)
    rs = (rs0, rs1, rs2, rs3)
    ws = (ws0, ws1, ws2, ws3)

    def c0_of(k):
        return pl.multiple_of(jnp.minimum((k * 32 + w) * 128, LASTC0), 128)

    def fire_read(k, b):
        pltpu.async_copy(tT.at[:, pl.ds(c0_of(k), 128)], blks[b], rs[b])

    def wait_read(k, b):
        pltpu.make_async_copy(
            tT.at[:, pl.ds(c0_of(k), 128)], blks[b], rs[b]
        ).wait()

    def fire_write(k, b):
        pltpu.async_copy(ts[b], P.at[pl.ds(c0_of(k), 128)], ws[b])

    def wait_write(k, b):
        pltpu.make_async_copy(ts[b], P.at[pl.ds(c0_of(k), 128)], ws[b]).wait()

    def outer(o, carry):
        for b in range(4):
            k = 4 * o + b

            @pl.when(k < NBLK)
            def _fire():
                fire_read(k, b)

            kp = k - 3
            bp = (b + 1) % 4

            @pl.when(jnp.logical_and(kp >= 0, kp < NBLK))
            def _retire():
                wait_read(kp, bp)

                @pl.when(kp >= 4)
                def _():
                    wait_write(kp - 4, bp)

                _a_transpose(blks[bp], ts[bp])
                fire_write(kp, bp)

        return carry

    lax.fori_loop(0, NBLK // 4 + 1, outer, 0)
    for q in range(4):
        k = NBLK - 4 + q
        wait_write(k, k % 4)

    @pl.when(w == 0)
    def _tail():
        pltpu.sync_copy(tail, tl_v)
        _a_transpose(tl_v, t0, nj=TAIL)
        pltpu.sync_copy(t0.at[pl.ds(0, TAIL)], P.at[pl.ds(7812 * 128, TAIL)])


PER_W = L


ROWS16 = None


def _b_transpose(g, t):
    iota, _ = _perms()

    @plsc.parallel_loop(0, 128, unroll=2)
    def _(i):
        bq = lax.shift_right_logical(i, 4)
        k = lax.bitwise_and(i, 15)
        b_vec = lax.bitwise_and(iota + k, 15) + bq * 16
        for eq in range(4):
            e_vec = iota + eq * 16
            v = plsc.load_gather(g, [b_vec, e_vec])
            plsc.store_scatter(t, [(e_vec * 128) + b_vec], v)


def _b_body(idx2, P2, out, slab, c0, c1, c2, c3, g0, g1, g2, g3, t0, t1,
            gs0, gs1, gs2, gs3, ss0, ss1):
    w = lax.axis_index("s") * NC + lax.axis_index("c")
    cs = (c0, c1, c2, c3)
    gs = (g0, g1, g2, g3)
    ts = (t0, t1)
    gsem = (gs0, gs1, gs2, gs3)
    ssem = (ss0, ss1)

    pltpu.sync_copy(idx2.at[pl.ds(w * PER_W, PER_W)], slab)
    bases = [_iota16(L, j8 * 16 * L) for j8 in range(8)]

    def assemble(l, b):
        for j8 in range(8):
            pos = bases[j8] + l
            v = plsc.load_gather(
                slab, [lax.shift_right_logical(pos, 7), lax.bitwise_and(pos, 127)]
            )
            cs[b][pl.ds(j8 * 16, 16)] = v

    def fire_gather(b):
        pltpu.async_copy(P2.at[cs[b]], gs[b], gsem[b])

    def wait_gather(b):
        pltpu.make_async_copy(P2.at[cs[b]], gs[b], gsem[b]).wait()

    def fire_stores(l, b):
        for e8 in range(8):
            pltpu.async_copy(
                ts[b].at[pl.ds(e8 * 1024, 1024)], out.at[l, e8, w], ssem[b]
            )

    def wait_stores(l, b):
        for e8 in range(8):
            pltpu.make_async_copy(
                ts[b].at[pl.ds(e8 * 1024, 1024)], out.at[l, e8, w], ssem[b]
            ).wait()

    def outer(o, carry):
        for b in range(4):
            l = 4 * o + b

            @pl.when(l < L)
            def _fire():
                assemble(l, b)
                fire_gather(b)

            lp = l - 3
            bp = (b + 1) % 4
            tp = (b + 1) % 2

            @pl.when(jnp.logical_and(lp >= 0, lp < L))
            def _retire():
                wait_gather(bp)

                @pl.when(lp >= 2)
                def _():
                    wait_stores(lp - 2, tp)

                _b_transpose(gs[bp], ts[tp])
                fire_stores(lp, tp)

        return carry

    lax.fori_loop(0, L // 4 + 1, outer, 0)
    wait_stores(L - 2, (L - 2) % 2)
    wait_stores(L - 1, (L - 1) % 2)


@jax.jit
def kernel(idx, table):
    mesh = plsc.VectorSubcoreMesh(
        core_axis_name="c", subcore_axis_name="s", num_cores=NC, num_subcores=NS
    )
    P = pl.kernel(
        _a_body,
        out_type=jax.ShapeDtypeStruct((VOCAB, 128), jnp.float32),
        mesh=mesh,
        scratch_types=(
            [pltpu.VMEM((64, 128), jnp.float32)] * 4
            + [pltpu.VMEM((128, 128), jnp.float32)] * 4
            + [pltpu.VMEM((64, TAIL), jnp.float32)]
            + [pltpu.SemaphoreType.DMA] * 8
        ),
        compiler_params=pltpu.CompilerParams(
            use_tc_tiling_on_sc=True, needs_layout_passes=False
        ),
    )(table.T, table[7812 * 128 :].T)

    P2 = P.reshape(2 * VOCAB, EMB)
    idx2 = idx.reshape(B * L // 128, 128).astype(jnp.int32) * 2
    outL = pl.kernel(
        _b_body,
        out_type=jax.ShapeDtypeStruct((L, 8, 32, 1024), jnp.float32),
        mesh=mesh,
        scratch_types=(
            [pltpu.VMEM((PER_W, 128), jnp.int32)]
            + [pltpu.VMEM((128,), jnp.int32)] * 4
            + [pltpu.VMEM((128, 64), jnp.float32)] * 4
            + [pltpu.VMEM((8192,), jnp.float32)] * 2
            + [pltpu.SemaphoreType.DMA] * 6
        ),
        compiler_params=pltpu.CompilerParams(
            use_tc_tiling_on_sc=False, needs_layout_passes=False
        ),
    )(idx2, P2)

    outL5 = outL.reshape(L, 8, 32, 8, 128)
    return outL5.transpose(2, 4, 0, 1, 3).reshape(B, L, EMB)


# speedup vs baseline: 6.0158x; 1.0016x over previous
import jax
import jax.numpy as jnp
from jax import lax
from jax.experimental import pallas as pl
from jax.experimental.pallas import tpu as pltpu
from jax.experimental.pallas import tpu_sc as plsc

VOCAB = 1000000
EMB = 64
B = 4096
L = 200

NC = 2
NS = 16
NW = NC * NS


NBLK = 248
LASTC0 = 999808
TAIL = VOCAB - 7812 * 128


def _iota16(mult, base):
    return lax.iota(jnp.int32, 16) * mult + base


def _perms():
    iota = lax.iota(jnp.int32, 16)
    return iota, [lax.bitwise_and(iota + k, 15) for k in range(16)]


def _a_transpose(blk, t, nj=128):
    iota, _ = _perms()

    @plsc.parallel_loop(0, (nj // 16) * 16, unroll=4)
    def _(i):
        jq = lax.shift_right_logical(i, 4)
        k = lax.bitwise_and(i, 15)
        j_vec = iota + jq * 16
        pv = lax.bitwise_and(iota + k, 15)
        for eq in range(4):
            e_vec = pv + eq * 16
            v = plsc.load_gather(blk, [e_vec, j_vec])
            plsc.store_scatter(t, [j_vec, e_vec], v)


def _a_body(tT, tail, P, blk0, blk1, blk2, blk3, t0, t1, t2, t3, tl_v,
            rs0, rs1, rs2, rs3, ws0, ws1, ws2, ws3):
    w = lax.axis_index("s") * NC + lax.axis_index("c")
    blks = (blk0, blk1, blk2, blk3)
    ts = (t0, t1, t2, t3)
    rs = (rs0, rs1, rs2, rs3)
    ws = (ws0, ws1, ws2, ws3)

    def c0_of(k):
        return pl.multiple_of(jnp.minimum((k * 32 + w) * 128, LASTC0), 128)

    def fire_read(k, b):
        pltpu.async_copy(tT.at[:, pl.ds(c0_of(k), 128)], blks[b], rs[b])

    def wait_read(k, b):
        pltpu.make_async_copy(
            tT.at[:, pl.ds(c0_of(k), 128)], blks[b], rs[b]
        ).wait()

    def fire_write(k, b):
        pltpu.async_copy(ts[b], P.at[pl.ds(c0_of(k), 128)], ws[b])

    def wait_write(k, b):
        pltpu.make_async_copy(ts[b], P.at[pl.ds(c0_of(k), 128)], ws[b]).wait()

    def outer(o, carry):
        for b in range(4):
            k = 4 * o + b

            @pl.when(k < NBLK)
            def _fire():
                fire_read(k, b)

            kp = k - 3
            bp = (b + 1) % 4

            @pl.when(jnp.logical_and(kp >= 0, kp < NBLK))
            def _retire():
                wait_read(kp, bp)

                @pl.when(kp >= 4)
                def _():
                    wait_write(kp - 4, bp)

                _a_transpose(blks[bp], ts[bp])
                fire_write(kp, bp)

        return carry

    lax.fori_loop(0, NBLK // 4 + 1, outer, 0)
    for q in range(4):
        k = NBLK - 4 + q
        wait_write(k, k % 4)

    @pl.when(w == 0)
    def _tail():
        pltpu.sync_copy(tail, tl_v)
        _a_transpose(tl_v, t0, nj=TAIL)
        pltpu.sync_copy(t0.at[pl.ds(0, TAIL)], P.at[pl.ds(7812 * 128, TAIL)])


PER_W = L


ROWS16 = None


def _b_transpose(g, t):
    iota, _ = _perms()

    @plsc.parallel_loop(0, 128, unroll=2)
    def _(i):
        bq = lax.shift_right_logical(i, 4)
        k = lax.bitwise_and(i, 15)
        b_vec = lax.bitwise_and(iota + k, 15) + bq * 16
        for eq in range(4):
            e_vec = iota + eq * 16
            v = plsc.load_gather(g, [b_vec, e_vec])
            plsc.store_scatter(t, [(e_vec * 128) + b_vec], v)


def _b_body(idx2, P2, out, slab, c0, c1, c2, c3, g0, g1, g2, g3, t0, t1,
            gs0, gs1, gs2, gs3, ss0, ss1):
    w = lax.axis_index("s") * NC + lax.axis_index("c")
    cs = (c0, c1, c2, c3)
    gs = (g0, g1, g2, g3)
    ts = (t0, t1)
    gsem = (gs0, gs1, gs2, gs3)
    ssem = (ss0, ss1)

    pltpu.sync_copy(idx2.at[pl.ds(w * PER_W, PER_W)], slab)
    bases = [_iota16(L, j8 * 16 * L) for j8 in range(8)]

    def assemble(l, b):
        for j8 in range(8):
            pos = bases[j8] + l
            v = plsc.load_gather(
                slab, [lax.shift_right_logical(pos, 7), lax.bitwise_and(pos, 127)]
            )
            cs[b][pl.ds(j8 * 16, 16)] = v

    def fire_gather(b):
        pltpu.async_copy(P2.at[cs[b]], gs[b], gsem[b])

    def wait_gather(b):
        pltpu.make_async_copy(P2.at[cs[b]], gs[b], gsem[b]).wait()

    def fire_stores(l, b):
        for e8 in range(8):
            pltpu.async_copy(
                ts[b].at[pl.ds(e8 * 1024, 1024)], out.at[l, e8, w], ssem[b]
            )

    def wait_stores(l, b):
        for e8 in range(8):
            pltpu.make_async_copy(
                ts[b].at[pl.ds(e8 * 1024, 1024)], out.at[l, e8, w], ssem[b]
            ).wait()

    def outer(o, carry):
        for b in range(4):
            l = 4 * o + b

            @pl.when(l < L)
            def _fire():
                assemble(l, b)
                fire_gather(b)

            lp = l - 3
            bp = (b + 1) % 4
            tp = (b + 1) % 2

            @pl.when(jnp.logical_and(lp >= 0, lp < L))
            def _retire():
                wait_gather(bp)

                @pl.when(lp >= 2)
                def _():
                    wait_stores(lp - 2, tp)

                _b_transpose(gs[bp], ts[tp])
                fire_stores(lp, tp)

        return carry

    lax.fori_loop(0, L // 4 + 1, outer, 0)
    wait_stores(L - 2, (L - 2) % 2)
    wait_stores(L - 1, (L - 1) % 2)


@jax.jit
def kernel(idx, table):
    mesh = plsc.VectorSubcoreMesh(
        core_axis_name="c", subcore_axis_name="s", num_cores=NC, num_subcores=NS
    )
    P = pl.kernel(
        _a_body,
        out_type=jax.ShapeDtypeStruct((VOCAB, 128), jnp.float32),
        mesh=mesh,
        scratch_types=(
            [pltpu.VMEM((64, 128), jnp.float32)] * 4
            + [pltpu.VMEM((128, 128), jnp.float32)] * 4
            + [pltpu.VMEM((64, TAIL), jnp.float32)]
            + [pltpu.SemaphoreType.DMA] * 8
        ),
        compiler_params=pltpu.CompilerParams(
            use_tc_tiling_on_sc=True, needs_layout_passes=False
        ),
    )(table.T, table[7812 * 128 :].T)

    P2 = P.reshape(2 * VOCAB, EMB)
    idx2 = idx.reshape(B * L // 128, 128).astype(jnp.int32) * 2
    outL = pl.kernel(
        _b_body,
        out_type=jax.ShapeDtypeStruct((L, 8, 32, 1024), jnp.float32),
        mesh=mesh,
        scratch_types=(
            [pltpu.VMEM((PER_W, 128), jnp.int32)]
            + [pltpu.VMEM((128,), jnp.int32)] * 4
            + [pltpu.VMEM((128, 64), jnp.float32)] * 4
            + [pltpu.VMEM((8192,), jnp.float32)] * 2
            + [pltpu.SemaphoreType.DMA] * 6
        ),
        compiler_params=pltpu.CompilerParams(
            use_tc_tiling_on_sc=False, needs_layout_passes=False
        ),
    )(idx2, P2)

    outL5 = outL.reshape(L, 8, 32, 8, 128)
    return outL5.transpose(2, 4, 0, 1, 3).reshape(B, L, EMB)


# speedup vs baseline: 6.0270x; 1.0018x over previous
import jax
import jax.numpy as jnp
from jax import lax
from jax.experimental import pallas as pl
from jax.experimental.pallas import tpu as pltpu
from jax.experimental.pallas import tpu_sc as plsc

VOCAB = 1000000
EMB = 64
B = 4096
L = 200

NC = 2
NS = 16
NW = NC * NS


NBLK = 248
LASTC0 = 999808
TAIL = VOCAB - 7812 * 128


def _iota16(mult, base):
    return lax.iota(jnp.int32, 16) * mult + base


def _perms():
    iota = lax.iota(jnp.int32, 16)
    return iota, [lax.bitwise_and(iota + k, 15) for k in range(16)]


def _a_transpose(blk, t, nj=128):
    iota, _ = _perms()

    @plsc.parallel_loop(0, (nj // 16) * 16, unroll=4)
    def _(i):
        jq = lax.shift_right_logical(i, 4)
        k = lax.bitwise_and(i, 15)
        j_vec = iota + jq * 16
        pv = lax.bitwise_and(iota + k, 15)
        for eq in range(4):
            e_vec = pv + eq * 16
            v = plsc.load_gather(blk, [e_vec, j_vec])
            plsc.store_scatter(t, [j_vec, e_vec], v)


def _a_body(tT, tail, P, blk0, blk1, blk2, blk3, t0, t1, t2, t3, tl_v,
            rs0, rs1, rs2, rs3, ws0, ws1, ws2, ws3):
    w = lax.axis_index("s") * NC + lax.axis_index("c")
    blks = (blk0, blk1, blk2, blk3)
    ts = (t0, t1, t2, t3)
    rs = (rs0, rs1, rs2, rs3)
    ws = (ws0, ws1, ws2, ws3)

    def c0_of(k):
        return pl.multiple_of(jnp.minimum((k * 32 + w) * 128, LASTC0), 128)

    def fire_read(k, b):
        pltpu.async_copy(tT.at[:, pl.ds(c0_of(k), 128)], blks[b], rs[b])

    def wait_read(k, b):
        pltpu.make_async_copy(
            tT.at[:, pl.ds(c0_of(k), 128)], blks[b], rs[b]
        ).wait()

    def fire_write(k, b):
        pltpu.async_copy(ts[b], P.at[pl.ds(c0_of(k), 128)], ws[b])

    def wait_write(k, b):
        pltpu.make_async_copy(ts[b], P.at[pl.ds(c0_of(k), 128)], ws[b]).wait()

    def outer(o, carry):
        for b in range(4):
            k = 4 * o + b

            @pl.when(k < NBLK)
            def _fire():
                fire_read(k, b)

            kp = k - 3
            bp = (b + 1) % 4

            @pl.when(jnp.logical_and(kp >= 0, kp < NBLK))
            def _retire():
                wait_read(kp, bp)

                @pl.when(kp >= 4)
                def _():
                    wait_write(kp - 4, bp)

                _a_transpose(blks[bp], ts[bp])
                fire_write(kp, bp)

        return carry

    lax.fori_loop(0, NBLK // 4 + 1, outer, 0)
    for q in range(4):
        k = NBLK - 4 + q
        wait_write(k, k % 4)

    @pl.when(w == 0)
    def _tail():
        pltpu.sync_copy(tail, tl_v)
        _a_transpose(tl_v, t0, nj=TAIL)
        pltpu.sync_copy(t0.at[pl.ds(0, TAIL)], P.at[pl.ds(7812 * 128, TAIL)])


PER_W = L


def _b_transpose(g, t):
    iota, _ = _perms()

    @plsc.parallel_loop(0, 128, unroll=2)
    def _(i):
        bq = lax.shift_right_logical(i, 4)
        k = lax.bitwise_and(i, 15)
        b_vec = lax.bitwise_and(iota + k, 15) + bq * 16
        for eq in range(4):
            e_vec = iota + eq * 16
            v = plsc.load_gather(g, [b_vec, e_vec])
            plsc.store_scatter(t, [(e_vec * 128) + b_vec], v)


def _b_body(idx2, P2, out, slab, c0, c1, c2, c3, g0, g1, g2, g3, t0, t1,
            gs0, gs1, gs2, gs3, ss0, ss1):
    w = lax.axis_index("s") * NC + lax.axis_index("c")
    cs = (c0, c1, c2, c3)
    gs = (g0, g1, g2, g3)
    ts = (t0, t1)
    gsem = (gs0, gs1, gs2, gs3)
    ssem = (ss0, ss1)

    pltpu.sync_copy(idx2.at[pl.ds(w * PER_W, PER_W)], slab)
    bases = [_iota16(L, j8 * 16 * L) for j8 in range(8)]

    def assemble(l, b):
        for j8 in range(8):
            pos = bases[j8] + l
            v = plsc.load_gather(
                slab, [lax.shift_right_logical(pos, 7), lax.bitwise_and(pos, 127)]
            )
            cs[b][pl.ds(j8 * 16, 16)] = v

    def fire_gather(b):
        pltpu.async_copy(P2.at[cs[b]], gs[b], gsem[b])

    def wait_gather(b):
        pltpu.make_async_copy(P2.at[cs[b]], gs[b], gsem[b]).wait()

    def fire_stores(l, b):
        for e8 in range(8):
            pltpu.async_copy(
                ts[b].at[pl.ds(e8 * 1024, 1024)], out.at[l, e8, w], ssem[b]
            )

    def wait_stores(l, b):
        for e8 in range(8):
            pltpu.make_async_copy(
                ts[b].at[pl.ds(e8 * 1024, 1024)], out.at[l, e8, w], ssem[b]
            ).wait()

    def outer(o, carry):
        for b in range(4):
            l = 4 * o + b

            @pl.when(l < L)
            def _fire():
                assemble(l, b)
                fire_gather(b)

            lp = l - 3
            bp = (b + 1) % 4
            tp = (b + 1) % 2

            @pl.when(jnp.logical_and(lp >= 0, lp < L))
            def _retire():
                wait_gather(bp)

                @pl.when(lp >= 2)
                def _():
                    wait_stores(lp - 2, tp)

                _b_transpose(gs[bp], ts[tp])
                fire_stores(lp, tp)

        return carry

    lax.fori_loop(0, L // 4 + 1, outer, 0)
    wait_stores(L - 2, (L - 2) % 2)
    wait_stores(L - 1, (L - 1) % 2)


@jax.jit
def kernel(idx, table):
    mesh = plsc.VectorSubcoreMesh(
        core_axis_name="c", subcore_axis_name="s", num_cores=NC, num_subcores=NS
    )
    P = pl.kernel(
        _a_body,
        out_type=jax.ShapeDtypeStruct((VOCAB, 128), jnp.float32),
        mesh=mesh,
        scratch_types=(
            [pltpu.VMEM((64, 128), jnp.float32)] * 4
            + [pltpu.VMEM((128, 128), jnp.float32)] * 4
            + [pltpu.VMEM((64, TAIL), jnp.float32)]
            + [pltpu.SemaphoreType.DMA] * 8
        ),
        compiler_params=pltpu.CompilerParams(
            use_tc_tiling_on_sc=True, needs_layout_passes=False
        ),
    )(table.T, table[7812 * 128 :].T)

    P2 = P.reshape(2 * VOCAB, EMB)
    idx2 = idx.reshape(B * L // 128, 128).astype(jnp.int32) * 2
    outL = pl.kernel(
        _b_body,
        out_type=jax.ShapeDtypeStruct((L, 8, 32, 1024), jnp.float32),
        mesh=mesh,
        scratch_types=(
            [pltpu.VMEM((PER_W, 128), jnp.int32)]
            + [pltpu.VMEM((128,), jnp.int32)] * 4
            + [pltpu.VMEM((128, 64), jnp.float32)] * 4
            + [pltpu.VMEM((8192,), jnp.float32)] * 2
            + [pltpu.SemaphoreType.DMA] * 6
        ),
        compiler_params=pltpu.CompilerParams(
            use_tc_tiling_on_sc=False, needs_layout_passes=False
        ),
    )(idx2, P2)

    outL5 = outL.reshape(L, 8, 32, 8, 128)
    return outL5.transpose(2, 4, 0, 1, 3).reshape(B, L, EMB)
